# initial kernel scaffold (unmeasured)
import jax
import jax.numpy as jnp
from jax import lax
from jax.experimental import pallas as pl
from jax.experimental.pallas import tpu as pltpu

N_DEV = 8
SQ = 2048
DH = 128
HQ_LOC = 8
D_MODEL = 1024
QT = 512
N_QT = SQ // QT
CHUNK = SQ // N_DEV
SCALE = 0.08838834764831843
BLK = 64


def _attn_body(x_ref, wq_ref, k_ref, v_ref, ctx_ref):
    qt = pl.program_id(1)
    q = jnp.dot(x_ref[...], wq_ref[...], preferred_element_type=jnp.float32)
    q = q.astype(jnp.bfloat16)
    k = k_ref[:, 0, :]
    s = lax.dot_general(
        q, k, (((1,), (1,)), ((), ())), preferred_element_type=jnp.float32
    )
    s = s * SCALE
    row = qt * QT + lax.broadcasted_iota(jnp.int32, (QT, SQ), 0)
    col = lax.broadcasted_iota(jnp.int32, (QT, SQ), 1)
    s = jnp.where((col // BLK) <= (row // BLK), s, -1e9)
    m = jnp.max(s, axis=1, keepdims=True)
    w = jnp.exp(s - m)
    w = w / jnp.sum(w, axis=1, keepdims=True)
    ctx = jnp.dot(
        w.astype(jnp.bfloat16), v_ref[:, 0, :], preferred_element_type=jnp.float32
    )
    ctx_ref[...] = ctx.astype(jnp.bfloat16)


def _ar_body(ctx_ref, wo_ref, out_ref, acc_ref, rs_buf,
             rs_ssem, rs_rsem, ag_ssem, ag_rsem):
    my = lax.axis_index("i")
    left = lax.rem(my - 1 + N_DEV, N_DEV)
    right = lax.rem(my + 1, N_DEV)

    barrier = pltpu.get_barrier_semaphore()
    for nbr in (left, right):
        pl.semaphore_signal(
            barrier, inc=1, device_id=(nbr,),
            device_id_type=pl.DeviceIdType.MESH,
        )
    pl.semaphore_wait(barrier, 2)

    acc_ref[...] = jnp.dot(
        ctx_ref[...], wo_ref[...], preferred_element_type=jnp.float32
    )

    for s in range(N_DEV - 1):
        send_idx = lax.rem(my - s + N_DEV, N_DEV)
        rdma = pltpu.make_async_remote_copy(
            src_ref=acc_ref.at[pl.ds(send_idx * CHUNK, CHUNK), :],
            dst_ref=rs_buf.at[s],
            send_sem=rs_ssem.at[s],
            recv_sem=rs_rsem.at[s],
            device_id=(right,),
            device_id_type=pl.DeviceIdType.MESH,
        )
        rdma.start()
        rdma.wait()
        recv_idx = lax.rem(my - s - 1 + N_DEV, N_DEV)
        rl = pl.ds(recv_idx * CHUNK, CHUNK)
        acc_ref[rl, :] = acc_ref[rl, :] + rs_buf[s]

    own = lax.rem(my + 1, N_DEV)
    ol = pl.ds(own * CHUNK, CHUNK)
    out_ref[ol, :] = acc_ref[ol, :]

    for s in range(N_DEV - 1):
        send_idx = lax.rem(my + 1 - s + N_DEV, N_DEV)
        sl = pl.ds(send_idx * CHUNK, CHUNK)
        rdma = pltpu.make_async_remote_copy(
            src_ref=out_ref.at[sl, :],
            dst_ref=out_ref.at[sl, :],
            send_sem=ag_ssem.at[s],
            recv_sem=ag_rsem.at[s],
            device_id=(right,),
            device_id_type=pl.DeviceIdType.MESH,
        )
        rdma.start()
        rdma.wait()


def kernel(x, Wq, K_ext, V_ext, Wo):
    my = lax.axis_index("i")
    x2 = x[0].astype(jnp.bfloat16)
    wq = Wq.astype(jnp.bfloat16)
    k = lax.dynamic_slice_in_dim(
        K_ext[0], my * HQ_LOC, HQ_LOC, axis=1
    ).astype(jnp.bfloat16)
    v = lax.dynamic_slice_in_dim(
        V_ext[0], my * HQ_LOC, HQ_LOC, axis=1
    ).astype(jnp.bfloat16)
    wo = Wo.astype(jnp.bfloat16)

    ctx = pl.pallas_call(
        _attn_body,
        grid=(HQ_LOC, N_QT),
        in_specs=[
            pl.BlockSpec((QT, D_MODEL), lambda h, qt: (qt, 0)),
            pl.BlockSpec((D_MODEL, DH), lambda h, qt: (0, h)),
            pl.BlockSpec((SQ, 1, DH), lambda h, qt: (0, h, 0)),
            pl.BlockSpec((SQ, 1, DH), lambda h, qt: (0, h, 0)),
        ],
        out_specs=pl.BlockSpec((QT, DH), lambda h, qt: (qt, h)),
        out_shape=jax.ShapeDtypeStruct((SQ, HQ_LOC * DH), jnp.bfloat16),
    )(x2, wq, k, v)

    out = pl.pallas_call(
        _ar_body,
        in_specs=[
            pl.BlockSpec(memory_space=pltpu.VMEM),
            pl.BlockSpec(memory_space=pltpu.VMEM),
        ],
        out_specs=pl.BlockSpec(memory_space=pltpu.VMEM),
        out_shape=jax.ShapeDtypeStruct((SQ, D_MODEL), jnp.float32),
        scratch_shapes=[
            pltpu.VMEM((SQ, D_MODEL), jnp.float32),
            pltpu.VMEM((N_DEV - 1, CHUNK, D_MODEL), jnp.float32),
            pltpu.SemaphoreType.DMA((N_DEV - 1,)),
            pltpu.SemaphoreType.DMA((N_DEV - 1,)),
            pltpu.SemaphoreType.DMA((N_DEV - 1,)),
            pltpu.SemaphoreType.DMA((N_DEV - 1,)),
        ],
        compiler_params=pltpu.CompilerParams(collective_id=0),
    )(ctx, wo)

    return out[None]


# baseline (device time: 329765 ns/iter reference)
import jax
import jax.numpy as jnp
from jax import lax
from jax.experimental import pallas as pl
from jax.experimental.pallas import tpu as pltpu

N_DEV = 8
SQ = 2048
DH = 128
HQ_LOC = 8
D_MODEL = 1024
QT = 512
N_QT = SQ // QT
CHUNK = SQ // N_DEV
SCALE = 0.08838834764831843
BLK = 64


def _attn_body(x_ref, wq_ref, k_ref, v_ref, ctx_ref):
    qt = pl.program_id(1)
    q = jnp.dot(x_ref[...], wq_ref[...], preferred_element_type=jnp.float32)
    q = q.astype(jnp.bfloat16)
    k = k_ref[...]
    s = lax.dot_general(
        q, k, (((1,), (1,)), ((), ())), preferred_element_type=jnp.float32
    )
    s = s * SCALE
    row = qt * QT + lax.broadcasted_iota(jnp.int32, (QT, SQ), 0)
    col = lax.broadcasted_iota(jnp.int32, (QT, SQ), 1)
    s = jnp.where((col // BLK) <= (row // BLK), s, -1e9)
    m = jnp.max(s, axis=1, keepdims=True)
    w = jnp.exp(s - m)
    w = w / jnp.sum(w, axis=1, keepdims=True)
    ctx = jnp.dot(
        w.astype(jnp.bfloat16), v_ref[...], preferred_element_type=jnp.float32
    )
    ctx_ref[...] = ctx.astype(jnp.bfloat16)


def _ar_body(ctx_ref, wo_ref, out_ref, acc_ref, rs_buf,
             rs_ssem, rs_rsem, ag_ssem, ag_rsem):
    my = lax.axis_index("i")
    left = lax.rem(my - 1 + N_DEV, N_DEV)
    right = lax.rem(my + 1, N_DEV)

    barrier = pltpu.get_barrier_semaphore()
    for nbr in (left, right):
        pl.semaphore_signal(
            barrier, inc=1, device_id=(nbr,),
            device_id_type=pl.DeviceIdType.MESH,
        )
    pl.semaphore_wait(barrier, 2)

    acc_ref[...] = jnp.dot(
        ctx_ref[...], wo_ref[...], preferred_element_type=jnp.float32
    )

    for s in range(N_DEV - 1):
        send_idx = lax.rem(my - s + N_DEV, N_DEV)
        rdma = pltpu.make_async_remote_copy(
            src_ref=acc_ref.at[pl.ds(send_idx * CHUNK, CHUNK), :],
            dst_ref=rs_buf.at[s],
            send_sem=rs_ssem.at[s],
            recv_sem=rs_rsem.at[s],
            device_id=(right,),
            device_id_type=pl.DeviceIdType.MESH,
        )
        rdma.start()
        rdma.wait()
        recv_idx = lax.rem(my - s - 1 + N_DEV, N_DEV)
        rl = pl.ds(recv_idx * CHUNK, CHUNK)
        acc_ref[rl, :] = acc_ref[rl, :] + rs_buf[s]

    own = lax.rem(my + 1, N_DEV)
    ol = pl.ds(own * CHUNK, CHUNK)
    out_ref[ol, :] = acc_ref[ol, :]

    for s in range(N_DEV - 1):
        send_idx = lax.rem(my + 1 - s + N_DEV, N_DEV)
        sl = pl.ds(send_idx * CHUNK, CHUNK)
        rdma = pltpu.make_async_remote_copy(
            src_ref=out_ref.at[sl, :],
            dst_ref=out_ref.at[sl, :],
            send_sem=ag_ssem.at[s],
            recv_sem=ag_rsem.at[s],
            device_id=(right,),
            device_id_type=pl.DeviceIdType.MESH,
        )
        rdma.start()
        rdma.wait()


def kernel(x, Wq, K_ext, V_ext, Wo):
    my = lax.axis_index("i")
    x2 = x[0].astype(jnp.bfloat16)
    wq = Wq.astype(jnp.bfloat16)
    k = lax.dynamic_slice_in_dim(
        K_ext[0], my * HQ_LOC, HQ_LOC, axis=1
    ).astype(jnp.bfloat16).reshape(SQ, HQ_LOC * DH)
    v = lax.dynamic_slice_in_dim(
        V_ext[0], my * HQ_LOC, HQ_LOC, axis=1
    ).astype(jnp.bfloat16).reshape(SQ, HQ_LOC * DH)
    wo = Wo.astype(jnp.bfloat16)

    ctx = pl.pallas_call(
        _attn_body,
        grid=(HQ_LOC, N_QT),
        in_specs=[
            pl.BlockSpec((QT, D_MODEL), lambda h, qt: (qt, 0)),
            pl.BlockSpec((D_MODEL, DH), lambda h, qt: (0, h)),
            pl.BlockSpec((SQ, DH), lambda h, qt: (0, h)),
            pl.BlockSpec((SQ, DH), lambda h, qt: (0, h)),
        ],
        out_specs=pl.BlockSpec((QT, DH), lambda h, qt: (qt, h)),
        out_shape=jax.ShapeDtypeStruct((SQ, HQ_LOC * DH), jnp.bfloat16),
    )(x2, wq, k, v)

    out = pl.pallas_call(
        _ar_body,
        in_specs=[
            pl.BlockSpec(memory_space=pltpu.VMEM),
            pl.BlockSpec(memory_space=pltpu.VMEM),
        ],
        out_specs=pl.BlockSpec(memory_space=pltpu.VMEM),
        out_shape=jax.ShapeDtypeStruct((SQ, D_MODEL), jnp.float32),
        scratch_shapes=[
            pltpu.VMEM((SQ, D_MODEL), jnp.float32),
            pltpu.VMEM((N_DEV - 1, CHUNK, D_MODEL), jnp.float32),
            pltpu.SemaphoreType.DMA((N_DEV - 1,)),
            pltpu.SemaphoreType.DMA((N_DEV - 1,)),
            pltpu.SemaphoreType.DMA((N_DEV - 1,)),
            pltpu.SemaphoreType.DMA((N_DEV - 1,)),
        ],
        compiler_params=pltpu.CompilerParams(collective_id=0),
    )(ctx, wo)

    return out[None]


# device time: 284156 ns/iter; 1.1605x vs baseline; 1.1605x over previous
import jax
import jax.numpy as jnp
from jax import lax
from jax.experimental import pallas as pl
from jax.experimental.pallas import tpu as pltpu

N_DEV = 8
SQ = 2048
DH = 128
HQ_LOC = 8
D_MODEL = 1024
QT = 512
KT = 512
N_QT = SQ // QT
N_KT = SQ // KT
CHUNK = SQ // N_DEV
HC = D_MODEL // 2
SCALE = 0.08838834764831843
BLK = 64
NEG = -1e9


def _attn_body(x_ref, wq_ref, k_ref, v_ref, ctx_ref,
               q_scr, acc_scr, m_scr, l_scr):
    h = pl.program_id(0)
    qt = pl.program_id(1)
    kt = pl.program_id(2)

    @pl.when(kt == 0)
    def _init():
        q = jnp.dot(x_ref[...], wq_ref[...],
                    preferred_element_type=jnp.float32)
        q_scr[...] = (q * SCALE).astype(jnp.bfloat16)
        acc_scr[...] = jnp.zeros((QT, DH), jnp.float32)
        m_scr[...] = jnp.full((QT, 128), NEG, jnp.float32)
        l_scr[...] = jnp.zeros((QT, 128), jnp.float32)

    @pl.when(kt <= qt)
    def _tile():
        s = lax.dot_general(
            q_scr[...], k_ref[...], (((1,), (1,)), ((), ())),
            preferred_element_type=jnp.float32,
        )
        row = qt * QT + lax.broadcasted_iota(jnp.int32, (QT, KT), 0)
        col = kt * KT + lax.broadcasted_iota(jnp.int32, (QT, KT), 1)
        s = jnp.where((col // BLK) <= (row // BLK), s, NEG)

        m_old = m_scr[...][:, 0:1]
        m_new = jnp.maximum(m_old, jnp.max(s, axis=1, keepdims=True))
        p = jnp.exp(s - m_new)
        alpha = jnp.exp(m_old - m_new)
        l_old = l_scr[...][:, 0:1]
        l_new = l_old * alpha + jnp.sum(p, axis=1, keepdims=True)
        acc_scr[...] = acc_scr[...] * alpha + jnp.dot(
            p.astype(jnp.bfloat16), v_ref[...],
            preferred_element_type=jnp.float32,
        )
        m_scr[...] = jnp.broadcast_to(m_new, (QT, 128))
        l_scr[...] = jnp.broadcast_to(l_new, (QT, 128))

    @pl.when(kt == N_KT - 1)
    def _fin():
        l = l_scr[...][:, 0:1]
        ctx_ref[...] = (acc_scr[...] / l).astype(jnp.bfloat16)


def _ar_body(ctx_ref, wo_ref, out_ref, acc_ref,
             rs_sbuf, rs_rbuf, ag_sbuf, ag_rbuf,
             rs_ssem, rs_rsem, ag_ssem, ag_rsem):
    my = lax.axis_index("i")
    left = lax.rem(my - 1 + N_DEV, N_DEV)
    right = lax.rem(my + 1, N_DEV)
    nbr = (right, left)

    barrier = pltpu.get_barrier_semaphore()
    for n in (left, right):
        pl.semaphore_signal(
            barrier, inc=1, device_id=(n,),
            device_id_type=pl.DeviceIdType.MESH,
        )
    pl.semaphore_wait(barrier, 2)

    acc_ref[...] = jnp.dot(
        ctx_ref[...], wo_ref[...], preferred_element_type=jnp.float32
    )

    for s in range(N_DEV - 1):
        rdmas = []
        for d in range(2):
            if d == 0:
                send_idx = lax.rem(my - s + N_DEV, N_DEV)
            else:
                send_idx = lax.rem(my + s, N_DEV)
            rows = pl.ds(send_idx * CHUNK, CHUNK)
            rs_sbuf[d] = acc_ref[rows, d * HC:(d + 1) * HC].astype(jnp.bfloat16)
            rdma = pltpu.make_async_remote_copy(
                src_ref=rs_sbuf.at[d],
                dst_ref=rs_rbuf.at[d, s],
                send_sem=rs_ssem.at[d, s],
                recv_sem=rs_rsem.at[d, s],
                device_id=(nbr[d],),
                device_id_type=pl.DeviceIdType.MESH,
            )
            rdma.start()
            rdmas.append(rdma)
        for d in range(2):
            rdmas[d].wait()
            if d == 0:
                recv_idx = lax.rem(my - s - 1 + N_DEV, N_DEV)
            else:
                recv_idx = lax.rem(my + s + 1, N_DEV)
            rows = pl.ds(recv_idx * CHUNK, CHUNK)
            acc_ref[rows, d * HC:(d + 1) * HC] = (
                acc_ref[rows, d * HC:(d + 1) * HC]
                + rs_rbuf[d, s].astype(jnp.float32)
            )

    own = (lax.rem(my + 1, N_DEV), left)
    for d in range(2):
        rows = pl.ds(own[d] * CHUNK, CHUNK)
        vals = acc_ref[rows, d * HC:(d + 1) * HC]
        out_ref[rows, d * HC:(d + 1) * HC] = vals
        ag_sbuf[d] = vals.astype(jnp.bfloat16)

    for s in range(N_DEV - 1):
        rdmas = []
        for d in range(2):
            src = ag_sbuf.at[d] if s == 0 else ag_rbuf.at[d, s - 1]
            rdma = pltpu.make_async_remote_copy(
                src_ref=src,
                dst_ref=ag_rbuf.at[d, s],
                send_sem=ag_ssem.at[d, s],
                recv_sem=ag_rsem.at[d, s],
                device_id=(nbr[d],),
                device_id_type=pl.DeviceIdType.MESH,
            )
            rdma.start()
            rdmas.append(rdma)
        for d in range(2):
            rdmas[d].wait()
            if d == 0:
                recv_idx = lax.rem(my - s + N_DEV, N_DEV)
            else:
                recv_idx = lax.rem(my + s, N_DEV)
            rows = pl.ds(recv_idx * CHUNK, CHUNK)
            out_ref[rows, d * HC:(d + 1) * HC] = ag_rbuf[d, s].astype(
                jnp.float32
            )


def kernel(x, Wq, K_ext, V_ext, Wo):
    my = lax.axis_index("i")
    x2 = x[0].astype(jnp.bfloat16)
    wq = Wq.astype(jnp.bfloat16)
    k = lax.dynamic_slice_in_dim(
        K_ext[0], my * HQ_LOC, HQ_LOC, axis=1
    ).astype(jnp.bfloat16).reshape(SQ, HQ_LOC * DH)
    v = lax.dynamic_slice_in_dim(
        V_ext[0], my * HQ_LOC, HQ_LOC, axis=1
    ).astype(jnp.bfloat16).reshape(SQ, HQ_LOC * DH)
    wo = Wo.astype(jnp.bfloat16)

    ctx = pl.pallas_call(
        _attn_body,
        grid=(HQ_LOC, N_QT, N_KT),
        in_specs=[
            pl.BlockSpec((QT, D_MODEL), lambda h, qt, kt: (qt, 0)),
            pl.BlockSpec((D_MODEL, DH), lambda h, qt, kt: (0, h)),
            pl.BlockSpec((KT, DH), lambda h, qt, kt: (kt, h)),
            pl.BlockSpec((KT, DH), lambda h, qt, kt: (kt, h)),
        ],
        out_specs=pl.BlockSpec((QT, DH), lambda h, qt, kt: (qt, h)),
        out_shape=jax.ShapeDtypeStruct((SQ, HQ_LOC * DH), jnp.bfloat16),
        scratch_shapes=[
            pltpu.VMEM((QT, DH), jnp.bfloat16),
            pltpu.VMEM((QT, DH), jnp.float32),
            pltpu.VMEM((QT, 128), jnp.float32),
            pltpu.VMEM((QT, 128), jnp.float32),
        ],
    )(x2, wq, k, v)

    out = pl.pallas_call(
        _ar_body,
        in_specs=[
            pl.BlockSpec(memory_space=pltpu.VMEM),
            pl.BlockSpec(memory_space=pltpu.VMEM),
        ],
        out_specs=pl.BlockSpec(memory_space=pltpu.VMEM),
        out_shape=jax.ShapeDtypeStruct((SQ, D_MODEL), jnp.float32),
        scratch_shapes=[
            pltpu.VMEM((SQ, D_MODEL), jnp.float32),
            pltpu.VMEM((2, CHUNK, HC), jnp.bfloat16),
            pltpu.VMEM((2, N_DEV - 1, CHUNK, HC), jnp.bfloat16),
            pltpu.VMEM((2, CHUNK, HC), jnp.bfloat16),
            pltpu.VMEM((2, N_DEV - 1, CHUNK, HC), jnp.bfloat16),
            pltpu.SemaphoreType.DMA((2, N_DEV - 1)),
            pltpu.SemaphoreType.DMA((2, N_DEV - 1)),
            pltpu.SemaphoreType.DMA((2, N_DEV - 1)),
            pltpu.SemaphoreType.DMA((2, N_DEV - 1)),
        ],
        compiler_params=pltpu.CompilerParams(collective_id=0),
    )(ctx, wo)

    return out[None]


# device time: 182809 ns/iter; 1.8039x vs baseline; 1.5544x over previous
import jax
import jax.numpy as jnp
from jax import lax
from jax.experimental import pallas as pl
from jax.experimental.pallas import tpu as pltpu

N_DEV = 8
SQ = 2048
DH = 128
HQ_LOC = 8
D_MODEL = 1024
QT = 512
KT = 512
N_QT = SQ // QT
N_KT = SQ // KT
CHUNK = SQ // N_DEV
HC = D_MODEL // 2
SCALE = 0.08838834764831843
BLK = 64
NEG = -1e9


def _make_attn_body(qt, n_kv):

    def _attn_body(x_ref, wq_ref, k_ref, v_ref, ctx_ref):
        q = jnp.dot(x_ref[...], wq_ref[...],
                    preferred_element_type=jnp.float32)
        q = (q * SCALE).astype(jnp.bfloat16)
        s = lax.dot_general(
            q, k_ref[...], (((1,), (1,)), ((), ())),
            preferred_element_type=jnp.float32,
        )
        row = qt * QT + lax.broadcasted_iota(jnp.int32, (QT, n_kv), 0)
        col = lax.broadcasted_iota(jnp.int32, (QT, n_kv), 1)
        s = jnp.where((col // BLK) <= (row // BLK), s, NEG)
        m = jnp.max(s, axis=1, keepdims=True)
        w = jnp.exp(s - m)
        w = w / jnp.sum(w, axis=1, keepdims=True)
        ctx_ref[...] = jnp.dot(
            w.astype(jnp.bfloat16), v_ref[...],
            preferred_element_type=jnp.float32,
        ).astype(jnp.bfloat16)

    return _attn_body


def _ar_body(ctx_ref, wo_ref, out_ref, acc_ref,
             rs_sbuf, rs_rbuf, ag_sbuf, ag_rbuf,
             rs_ssem, rs_rsem, ag_ssem, ag_rsem):
    my = lax.axis_index("i")
    left = lax.rem(my - 1 + N_DEV, N_DEV)
    right = lax.rem(my + 1, N_DEV)
    nbr = (right, left)

    barrier = pltpu.get_barrier_semaphore()
    for n in (left, right):
        pl.semaphore_signal(
            barrier, inc=1, device_id=(n,),
            device_id_type=pl.DeviceIdType.MESH,
        )
    pl.semaphore_wait(barrier, 2)

    acc_ref[...] = jnp.dot(
        ctx_ref[...], wo_ref[...], preferred_element_type=jnp.float32
    )

    for s in range(N_DEV - 1):
        rdmas = []
        for d in range(2):
            if d == 0:
                send_idx = lax.rem(my - s + N_DEV, N_DEV)
            else:
                send_idx = lax.rem(my + s, N_DEV)
            rows = pl.ds(send_idx * CHUNK, CHUNK)
            rs_sbuf[d] = acc_ref[rows, d * HC:(d + 1) * HC].astype(jnp.bfloat16)
            rdma = pltpu.make_async_remote_copy(
                src_ref=rs_sbuf.at[d],
                dst_ref=rs_rbuf.at[d, s],
                send_sem=rs_ssem.at[d, s],
                recv_sem=rs_rsem.at[d, s],
                device_id=(nbr[d],),
                device_id_type=pl.DeviceIdType.MESH,
            )
            rdma.start()
            rdmas.append(rdma)
        for d in range(2):
            rdmas[d].wait()
            if d == 0:
                recv_idx = lax.rem(my - s - 1 + N_DEV, N_DEV)
            else:
                recv_idx = lax.rem(my + s + 1, N_DEV)
            rows = pl.ds(recv_idx * CHUNK, CHUNK)
            acc_ref[rows, d * HC:(d + 1) * HC] = (
                acc_ref[rows, d * HC:(d + 1) * HC]
                + rs_rbuf[d, s].astype(jnp.float32)
            )

    own = (lax.rem(my + 1, N_DEV), left)
    for d in range(2):
        rows = pl.ds(own[d] * CHUNK, CHUNK)
        vals = acc_ref[rows, d * HC:(d + 1) * HC]
        out_ref[rows, d * HC:(d + 1) * HC] = vals
        ag_sbuf[d] = vals.astype(jnp.bfloat16)

    for s in range(N_DEV - 1):
        rdmas = []
        for d in range(2):
            src = ag_sbuf.at[d] if s == 0 else ag_rbuf.at[d, s - 1]
            rdma = pltpu.make_async_remote_copy(
                src_ref=src,
                dst_ref=ag_rbuf.at[d, s],
                send_sem=ag_ssem.at[d, s],
                recv_sem=ag_rsem.at[d, s],
                device_id=(nbr[d],),
                device_id_type=pl.DeviceIdType.MESH,
            )
            rdma.start()
            rdmas.append(rdma)
        for d in range(2):
            rdmas[d].wait()
            if d == 0:
                recv_idx = lax.rem(my - s + N_DEV, N_DEV)
            else:
                recv_idx = lax.rem(my + s, N_DEV)
            rows = pl.ds(recv_idx * CHUNK, CHUNK)
            out_ref[rows, d * HC:(d + 1) * HC] = ag_rbuf[d, s].astype(
                jnp.float32
            )


def kernel(x, Wq, K_ext, V_ext, Wo):
    my = lax.axis_index("i")
    x2 = x[0].astype(jnp.bfloat16)
    wq = Wq.astype(jnp.bfloat16)
    k = lax.dynamic_slice_in_dim(
        K_ext[0], my * HQ_LOC, HQ_LOC, axis=1
    ).astype(jnp.bfloat16).reshape(SQ, HQ_LOC * DH)
    v = lax.dynamic_slice_in_dim(
        V_ext[0], my * HQ_LOC, HQ_LOC, axis=1
    ).astype(jnp.bfloat16).reshape(SQ, HQ_LOC * DH)
    wo = Wo.astype(jnp.bfloat16)

    tiles = []
    for qt in range(N_QT):
        n_kv = (qt + 1) * KT
        tiles.append(
            pl.pallas_call(
                _make_attn_body(qt, n_kv),
                grid=(HQ_LOC,),
                in_specs=[
                    pl.BlockSpec((QT, D_MODEL), lambda h, _qt=qt: (_qt, 0)),
                    pl.BlockSpec((D_MODEL, DH), lambda h: (0, h)),
                    pl.BlockSpec((n_kv, DH), lambda h: (0, h)),
                    pl.BlockSpec((n_kv, DH), lambda h: (0, h)),
                ],
                out_specs=pl.BlockSpec((QT, DH), lambda h: (0, h)),
                out_shape=jax.ShapeDtypeStruct(
                    (QT, HQ_LOC * DH), jnp.bfloat16
                ),
            )(x2, wq, k, v)
        )
    ctx = jnp.concatenate(tiles, axis=0)

    out = pl.pallas_call(
        _ar_body,
        in_specs=[
            pl.BlockSpec(memory_space=pltpu.VMEM),
            pl.BlockSpec(memory_space=pltpu.VMEM),
        ],
        out_specs=pl.BlockSpec(memory_space=pltpu.VMEM),
        out_shape=jax.ShapeDtypeStruct((SQ, D_MODEL), jnp.float32),
        scratch_shapes=[
            pltpu.VMEM((SQ, D_MODEL), jnp.float32),
            pltpu.VMEM((2, CHUNK, HC), jnp.bfloat16),
            pltpu.VMEM((2, N_DEV - 1, CHUNK, HC), jnp.bfloat16),
            pltpu.VMEM((2, CHUNK, HC), jnp.bfloat16),
            pltpu.VMEM((2, N_DEV - 1, CHUNK, HC), jnp.bfloat16),
            pltpu.SemaphoreType.DMA((2, N_DEV - 1)),
            pltpu.SemaphoreType.DMA((2, N_DEV - 1)),
            pltpu.SemaphoreType.DMA((2, N_DEV - 1)),
            pltpu.SemaphoreType.DMA((2, N_DEV - 1)),
        ],
        compiler_params=pltpu.CompilerParams(collective_id=0),
    )(ctx, wo)

    return out[None]


# device time: 112512 ns/iter; 2.9309x vs baseline; 1.6248x over previous
import jax
import jax.numpy as jnp
from jax import lax
from jax.experimental import pallas as pl
from jax.experimental.pallas import tpu as pltpu

N_DEV = 8
SQ = 2048
DH = 128
HQ_LOC = 8
D_MODEL = 1024
QT = 512
KT = 512
N_QT = SQ // QT
SUB = QT // N_DEV
SCALE = 0.08838834764831843
BLK = 64
NEG = -1e9
_MESH = pl.DeviceIdType.MESH


def _fused_body(x_ref, wq_ref, k_ref, v_ref, wo_ref, out_ref,
                ctx_scr, tsend, ag_sbuf, rs_rbuf, ag_rbuf,
                rs_ssem, rs_rsem, ag_ssem, ag_rsem):
    my = lax.axis_index("i")

    barrier = pltpu.get_barrier_semaphore()
    for p in range(1, N_DEV):
        peer = lax.rem(my + p, N_DEV)
        pl.semaphore_signal(barrier, inc=1, device_id=(peer,),
                            device_id_type=_MESH)
    pl.semaphore_wait(barrier, N_DEV - 1)

    def compute_tile(t):
        n_kv = (t + 1) * KT
        xt = x_ref[t * QT:(t + 1) * QT, :]
        for h in range(HQ_LOC):
            c0, c1 = h * DH, (h + 1) * DH
            q = jnp.dot(xt, wq_ref[:, c0:c1],
                        preferred_element_type=jnp.float32)
            q = (q * SCALE).astype(jnp.bfloat16)
            s = lax.dot_general(
                q, k_ref[0:n_kv, c0:c1], (((1,), (1,)), ((), ())),
                preferred_element_type=jnp.float32,
            )
            row = t * QT + lax.broadcasted_iota(jnp.int32, (QT, n_kv), 0)
            col = lax.broadcasted_iota(jnp.int32, (QT, n_kv), 1)
            s = jnp.where((col // BLK) <= (row // BLK), s, NEG)
            m = jnp.max(s, axis=1, keepdims=True)
            w = jnp.exp(s - m)
            w = w / jnp.sum(w, axis=1, keepdims=True)
            ctx_scr[:, c0:c1] = jnp.dot(
                w.astype(jnp.bfloat16), v_ref[0:n_kv, c0:c1],
                preferred_element_type=jnp.float32,
            ).astype(jnp.bfloat16)
        partial = jnp.dot(ctx_scr[...], wo_ref[...],
                          preferred_element_type=jnp.float32)
        tsend[t % 2] = partial.astype(jnp.bfloat16)

    def rs_descs(t):
        descs = []
        for j in range(N_DEV):
            q = lax.rem(my + j, N_DEV)
            descs.append(pltpu.make_async_remote_copy(
                src_ref=tsend.at[t % 2].at[pl.ds(q * SUB, SUB), :],
                dst_ref=rs_rbuf.at[t].at[pl.ds(my * SUB, SUB), :],
                send_sem=rs_ssem.at[t % 2, j],
                recv_sem=rs_rsem.at[t, my],
                device_id=(q,), device_id_type=_MESH,
            ))
        return descs

    def ag_descs(t):
        descs = []
        for j in range(N_DEV):
            q = lax.rem(my + j, N_DEV)
            descs.append(pltpu.make_async_remote_copy(
                src_ref=ag_sbuf.at[t % 2],
                dst_ref=ag_rbuf.at[t].at[pl.ds(my * SUB, SUB), :],
                send_sem=ag_ssem.at[t % 2, j],
                recv_sem=ag_rsem.at[t, my],
                device_id=(q,), device_id_type=_MESH,
            ))
        return descs

    def wait_recvs(t, rbuf, rsem):
        for s in range(N_DEV):
            desc = pltpu.make_async_remote_copy(
                src_ref=tsend.at[0].at[pl.ds(0, SUB), :],
                dst_ref=rbuf.at[t].at[pl.ds(s * SUB, SUB), :],
                send_sem=rs_ssem.at[0, 0],
                recv_sem=rsem.at[t, s],
                device_id=(my,), device_id_type=_MESH,
            )
            desc.wait_recv()

    rs_inflight = {}
    ag_inflight = {}

    def finish_rs(t):
        wait_recvs(t, rs_rbuf, rs_rsem)
        red = jnp.zeros((SUB, D_MODEL), jnp.float32)
        for s in range(N_DEV):
            red = red + rs_rbuf[t, s * SUB:(s + 1) * SUB, :].astype(
                jnp.float32)
        ag_sbuf[t % 2] = red.astype(jnp.bfloat16)
        ag_inflight[t] = ag_descs(t)
        for d in ag_inflight[t]:
            d.start()
        for d in rs_inflight[t]:
            d.wait_send()

    def finish_ag(t):
        wait_recvs(t, ag_rbuf, ag_rsem)
        out_ref[t * QT:(t + 1) * QT, :] = ag_rbuf[t].astype(jnp.float32)
        for d in ag_inflight[t]:
            d.wait_send()

    for t in range(N_QT):
        compute_tile(t)
        rs_inflight[t] = rs_descs(t)
        for d in rs_inflight[t]:
            d.start()
        if t >= 1:
            finish_rs(t - 1)
        if t >= 2:
            finish_ag(t - 2)
    finish_rs(N_QT - 1)
    finish_ag(N_QT - 2)
    finish_ag(N_QT - 1)


def kernel(x, Wq, K_ext, V_ext, Wo):
    my = lax.axis_index("i")
    x2 = x[0].astype(jnp.bfloat16)
    wq = Wq.astype(jnp.bfloat16)
    k = lax.dynamic_slice_in_dim(
        K_ext[0], my * HQ_LOC, HQ_LOC, axis=1
    ).astype(jnp.bfloat16).reshape(SQ, HQ_LOC * DH)
    v = lax.dynamic_slice_in_dim(
        V_ext[0], my * HQ_LOC, HQ_LOC, axis=1
    ).astype(jnp.bfloat16).reshape(SQ, HQ_LOC * DH)
    wo = Wo.astype(jnp.bfloat16)

    out = pl.pallas_call(
        _fused_body,
        in_specs=[pl.BlockSpec(memory_space=pltpu.VMEM)] * 5,
        out_specs=pl.BlockSpec(memory_space=pltpu.VMEM),
        out_shape=jax.ShapeDtypeStruct((SQ, D_MODEL), jnp.float32),
        scratch_shapes=[
            pltpu.VMEM((QT, D_MODEL), jnp.bfloat16),
            pltpu.VMEM((2, QT, D_MODEL), jnp.bfloat16),
            pltpu.VMEM((2, SUB, D_MODEL), jnp.bfloat16),
            pltpu.VMEM((N_QT, QT, D_MODEL), jnp.bfloat16),
            pltpu.VMEM((N_QT, QT, D_MODEL), jnp.bfloat16),
            pltpu.SemaphoreType.DMA((2, N_DEV)),
            pltpu.SemaphoreType.DMA((N_QT, N_DEV)),
            pltpu.SemaphoreType.DMA((2, N_DEV)),
            pltpu.SemaphoreType.DMA((N_QT, N_DEV)),
        ],
        compiler_params=pltpu.CompilerParams(collective_id=0),
    )(x2, wq, k, v, wo)

    return out[None]
